# baseline (device time: 46640 ns/iter reference)
import jax
import jax.numpy as jnp
from jax import lax
from jax.experimental import pallas as pl
from jax.experimental.pallas import tpu as pltpu

N_DEV = 4
SUB = 4
NSLOTS = (N_DEV - 1) * SUB


def kernel(x, W1, W2):
    m, _ = x.shape
    d = W1.shape[1]
    n = W2.shape[1]
    chunk = m // N_DEV
    half = chunk // 2
    sub = half // SUB
    nslots = NSLOTS

    def body(x_ref, w1_ref, w2_ref, out_ref, h_ref,
             recv_a, recv_b,
             rsa_send, rsa_recv, rsb_send, rsb_recv,
             aga_send, aga_recv, agb_send, agb_recv):
        my = lax.axis_index("i")
        left = (my + N_DEV - 1) % N_DEV
        right = (my + 1) % N_DEV

        def row_a(c, k=0):
            return ((c % N_DEV) * chunk) + k * sub

        def row_b(c, k=0):
            return ((c % N_DEV) * chunk) + half + k * sub

        def gemm1_tile(row, rows=half):
            h_ref[pl.ds(row, rows), :] = jnp.dot(
                x_ref[pl.ds(row, rows), :], w1_ref[...],
                preferred_element_type=jnp.float32,
            )

        def gemm2_piece(row):
            out_ref[pl.ds(row, sub), :] = jnp.dot(
                h_ref[pl.ds(row, sub), :], w2_ref[...],
                preferred_element_type=jnp.float32,
            )

        def send(src_row, dst_dev, sems_s, sems_r, slot, recvbuf, dst_row):
            rdma = pltpu.make_async_remote_copy(
                src_ref=h_ref.at[pl.ds(src_row, sub)],
                dst_ref=(recvbuf.at[slot] if recvbuf is not None
                         else h_ref.at[pl.ds(dst_row, sub)]),
                send_sem=sems_s.at[slot],
                recv_sem=sems_r.at[slot],
                device_id=(dst_dev,),
                device_id_type=pl.DeviceIdType.MESH,
            )
            rdma.start()
            return rdma

        barrier_sem = pltpu.get_barrier_semaphore()
        for nbr in [left, right]:
            pl.semaphore_signal(
                barrier_sem, inc=1,
                device_id=(nbr,), device_id_type=pl.DeviceIdType.MESH,
            )
        pl.semaphore_wait(barrier_sem, 2)

        ra = [None] * nslots
        rb = [None] * nslots
        for k in range(SUB):
            gemm1_tile(row_a(my + 3, k), sub)
            ra[k] = send(row_a(my + 3, k), right,
                         rsa_send, rsa_recv, k, recv_a, 0)
            gemm1_tile(row_b(my + 1, k), sub)
            rb[k] = send(row_b(my + 1, k), left,
                         rsb_send, rsb_recv, k, recv_b, 0)
        gemm1_tile(row_a(my + 2))
        gemm1_tile(row_b(my + 2))
        gemm1_tile(row_a(my + 1))
        gemm1_tile(row_b(my + 3))
        gemm1_tile(row_a(my))
        gemm1_tile(row_b(my))

        ga = [None] * nslots
        gb = [None] * nslots
        for s in range(N_DEV - 1):
            c_ra = my + 2 * N_DEV - 2 - s
            c_rb = my + 2 + s
            for k in range(SUB):
                slot = s * SUB + k
                ra[slot].wait()
                h_ref[pl.ds(row_a(c_ra, k), sub), :] += recv_a[slot]
                if s < N_DEV - 2:
                    ra[slot + SUB] = send(row_a(c_ra, k), right, rsa_send,
                                          rsa_recv, slot + SUB, recv_a, 0)
                else:
                    ga[k] = send(row_a(my, k), right,
                                 aga_send, aga_recv, k, None, row_a(my, k))
                rb[slot].wait()
                h_ref[pl.ds(row_b(c_rb, k), sub), :] += recv_b[slot]
                if s < N_DEV - 2:
                    rb[slot + SUB] = send(row_b(c_rb, k), left, rsb_send,
                                          rsb_recv, slot + SUB, recv_b, 0)
                else:
                    gb[k] = send(row_b(my, k), left,
                                 agb_send, agb_recv, k, None, row_b(my, k))

        for k in range(SUB):
            gemm2_piece(row_a(my, k))
            gemm2_piece(row_b(my, k))

        for hh in range(N_DEV - 1):
            c_ga = my + 2 * N_DEV - 1 - hh
            c_gb = my + 1 + hh
            for k in range(SUB):
                slot = hh * SUB + k
                ga[slot].wait()
                if hh < N_DEV - 2:
                    nslot = slot + SUB
                    ga[nslot] = send(row_a(c_ga, k), right, aga_send,
                                     aga_recv, nslot, None, row_a(c_ga, k))
                gemm2_piece(row_a(c_ga, k))
                gb[slot].wait()
                if hh < N_DEV - 2:
                    nslot = slot + SUB
                    gb[nslot] = send(row_b(c_gb, k), left, agb_send,
                                     agb_recv, nslot, None, row_b(c_gb, k))
                gemm2_piece(row_b(c_gb, k))

    return pl.pallas_call(
        body,
        out_shape=jax.ShapeDtypeStruct((m, n), jnp.float32),
        in_specs=[
            pl.BlockSpec(memory_space=pltpu.VMEM),
            pl.BlockSpec(memory_space=pltpu.VMEM),
            pl.BlockSpec(memory_space=pltpu.VMEM),
        ],
        out_specs=pl.BlockSpec(memory_space=pltpu.VMEM),
        scratch_shapes=[
            pltpu.VMEM((m, d), jnp.float32),
            pltpu.VMEM((NSLOTS, half // SUB, d), jnp.float32),
            pltpu.VMEM((NSLOTS, half // SUB, d), jnp.float32),
            pltpu.SemaphoreType.DMA((NSLOTS,)),
            pltpu.SemaphoreType.DMA((NSLOTS,)),
            pltpu.SemaphoreType.DMA((NSLOTS,)),
            pltpu.SemaphoreType.DMA((NSLOTS,)),
            pltpu.SemaphoreType.DMA((NSLOTS,)),
            pltpu.SemaphoreType.DMA((NSLOTS,)),
            pltpu.SemaphoreType.DMA((NSLOTS,)),
            pltpu.SemaphoreType.DMA((NSLOTS,)),
        ],
        compiler_params=pltpu.CompilerParams(collective_id=0),
    )(x, W1, W2)


# device time: 30523 ns/iter; 1.5280x vs baseline; 1.5280x over previous
import jax
import jax.numpy as jnp
from jax import lax
from jax.experimental import pallas as pl
from jax.experimental.pallas import tpu as pltpu

N_DEV = 4
SUB = 2
NSLOTS = (N_DEV - 1) * SUB


def kernel(x, W1, W2):
    m, _ = x.shape
    d = W1.shape[1]
    n = W2.shape[1]
    chunk = m // N_DEV
    half = chunk // 2
    sub = half // SUB
    nslots = NSLOTS

    def body(x_ref, w1_ref, w2_ref, out_ref, h_ref, g_ref, w2b_ref,
             sbuf_a, sbuf_b, recv_a, recv_b,
             rsa_send, rsa_recv, rsb_send, rsb_recv,
             aga_send, aga_recv, agb_send, agb_recv):
        my = lax.axis_index("i")
        left = (my + N_DEV - 1) % N_DEV
        right = (my + 1) % N_DEV

        def row_a(c, k=0):
            return ((c % N_DEV) * chunk) + k * sub

        def row_b(c, k=0):
            return ((c % N_DEV) * chunk) + half + k * sub

        def gemm1_tile(row, rows=half):
            h_ref[pl.ds(row, rows), :] = jnp.dot(
                x_ref[pl.ds(row, rows), :], w1_ref[...],
                preferred_element_type=jnp.float32,
            )

        def gemm2_piece(row):
            out_ref[pl.ds(row, sub), :] = jnp.dot(
                g_ref[pl.ds(row, sub), :], w2b_ref[...],
                preferred_element_type=jnp.float32,
            )

        def rs_send(buf, row, slot, dst_dev, sems_s, sems_r, rbuf):
            buf[slot, :, :] = h_ref[pl.ds(row, sub), :].astype(jnp.bfloat16)
            rdma = pltpu.make_async_remote_copy(
                src_ref=buf.at[slot],
                dst_ref=rbuf.at[slot],
                send_sem=sems_s.at[slot],
                recv_sem=sems_r.at[slot],
                device_id=(dst_dev,),
                device_id_type=pl.DeviceIdType.MESH,
            )
            rdma.start()
            return rdma

        def ag_send(row, slot, dst_dev, sems_s, sems_r):
            rdma = pltpu.make_async_remote_copy(
                src_ref=g_ref.at[pl.ds(row, sub)],
                dst_ref=g_ref.at[pl.ds(row, sub)],
                send_sem=sems_s.at[slot],
                recv_sem=sems_r.at[slot],
                device_id=(dst_dev,),
                device_id_type=pl.DeviceIdType.MESH,
            )
            rdma.start()
            return rdma

        barrier_sem = pltpu.get_barrier_semaphore()
        for nbr in [left, right]:
            pl.semaphore_signal(
                barrier_sem, inc=1,
                device_id=(nbr,), device_id_type=pl.DeviceIdType.MESH,
            )
        w2b_ref[...] = w2_ref[...].astype(jnp.bfloat16)
        gemm1_tile(row_a(my + 3, 0), sub)
        pl.semaphore_wait(barrier_sem, 2)

        ra = [None] * nslots
        rb = [None] * nslots
        ra[0] = rs_send(sbuf_a, row_a(my + 3, 0), 0, right,
                        rsa_send, rsa_recv, recv_a)
        gemm1_tile(row_b(my + 1, 0), sub)
        rb[0] = rs_send(sbuf_b, row_b(my + 1, 0), 0, left,
                        rsb_send, rsb_recv, recv_b)
        gemm1_tile(row_a(my + 3, 1), sub)
        ra[1] = rs_send(sbuf_a, row_a(my + 3, 1), 1, right,
                        rsa_send, rsa_recv, recv_a)
        gemm1_tile(row_b(my + 1, 1), sub)
        rb[1] = rs_send(sbuf_b, row_b(my + 1, 1), 1, left,
                        rsb_send, rsb_recv, recv_b)
        gemm1_tile(row_a(my + 2))
        gemm1_tile(row_b(my + 2))
        gemm1_tile(row_a(my + 1))
        gemm1_tile(row_b(my + 3))
        gemm1_tile(row_a(my))
        gemm1_tile(row_b(my))

        ga = [None] * nslots
        gb = [None] * nslots
        for s in range(N_DEV - 1):
            c_ra = my + 2 * N_DEV - 2 - s
            c_rb = my + 2 + s
            for k in range(SUB):
                slot = s * SUB + k
                ra[slot].wait()
                h_ref[pl.ds(row_a(c_ra, k), sub), :] += (
                    recv_a[slot].astype(jnp.float32))
                if s < N_DEV - 2:
                    ra[slot + SUB] = rs_send(
                        sbuf_a, row_a(c_ra, k), slot + SUB, right,
                        rsa_send, rsa_recv, recv_a)
                else:
                    g_ref[pl.ds(row_a(my, k), sub), :] = (
                        h_ref[pl.ds(row_a(my, k), sub), :]
                        .astype(jnp.bfloat16))
                    ga[k] = ag_send(row_a(my, k), k, right,
                                    aga_send, aga_recv)
                rb[slot].wait()
                h_ref[pl.ds(row_b(c_rb, k), sub), :] += (
                    recv_b[slot].astype(jnp.float32))
                if s < N_DEV - 2:
                    rb[slot + SUB] = rs_send(
                        sbuf_b, row_b(c_rb, k), slot + SUB, left,
                        rsb_send, rsb_recv, recv_b)
                else:
                    g_ref[pl.ds(row_b(my, k), sub), :] = (
                        h_ref[pl.ds(row_b(my, k), sub), :]
                        .astype(jnp.bfloat16))
                    gb[k] = ag_send(row_b(my, k), k, left,
                                    agb_send, agb_recv)

        for k in range(SUB):
            gemm2_piece(row_a(my, k))
            gemm2_piece(row_b(my, k))

        for hh in range(N_DEV - 1):
            c_ga = my + 2 * N_DEV - 1 - hh
            c_gb = my + 1 + hh
            for k in range(SUB):
                slot = hh * SUB + k
                ga[slot].wait()
                if hh < N_DEV - 2:
                    ga[slot + SUB] = ag_send(row_a(c_ga, k), slot + SUB,
                                             right, aga_send, aga_recv)
                gemm2_piece(row_a(c_ga, k))
                gb[slot].wait()
                if hh < N_DEV - 2:
                    gb[slot + SUB] = ag_send(row_b(c_gb, k), slot + SUB,
                                             left, agb_send, agb_recv)
                gemm2_piece(row_b(c_gb, k))

    return pl.pallas_call(
        body,
        out_shape=jax.ShapeDtypeStruct((m, n), jnp.float32),
        in_specs=[
            pl.BlockSpec(memory_space=pltpu.VMEM),
            pl.BlockSpec(memory_space=pltpu.VMEM),
            pl.BlockSpec(memory_space=pltpu.VMEM),
        ],
        out_specs=pl.BlockSpec(memory_space=pltpu.VMEM),
        scratch_shapes=[
            pltpu.VMEM((m, d), jnp.float32),
            pltpu.VMEM((m, d), jnp.bfloat16),
            pltpu.VMEM((d, n), jnp.bfloat16),
            pltpu.VMEM((NSLOTS, half // SUB, d), jnp.bfloat16),
            pltpu.VMEM((NSLOTS, half // SUB, d), jnp.bfloat16),
            pltpu.VMEM((NSLOTS, half // SUB, d), jnp.bfloat16),
            pltpu.VMEM((NSLOTS, half // SUB, d), jnp.bfloat16),
            pltpu.SemaphoreType.DMA((NSLOTS,)),
            pltpu.SemaphoreType.DMA((NSLOTS,)),
            pltpu.SemaphoreType.DMA((NSLOTS,)),
            pltpu.SemaphoreType.DMA((NSLOTS,)),
            pltpu.SemaphoreType.DMA((NSLOTS,)),
            pltpu.SemaphoreType.DMA((NSLOTS,)),
            pltpu.SemaphoreType.DMA((NSLOTS,)),
            pltpu.SemaphoreType.DMA((NSLOTS,)),
        ],
        compiler_params=pltpu.CompilerParams(collective_id=0),
    )(x, W1, W2)


# device time: 29591 ns/iter; 1.5762x vs baseline; 1.0315x over previous
import jax
import jax.numpy as jnp
from jax import lax
from jax.experimental import pallas as pl
from jax.experimental.pallas import tpu as pltpu

N_DEV = 4
SUB = 4
NSLOTS = (N_DEV - 1) * SUB


def kernel(x, W1, W2):
    m, _ = x.shape
    d = W1.shape[1]
    n = W2.shape[1]
    chunk = m // N_DEV
    half = chunk // 2
    sub = half // SUB
    nslots = NSLOTS

    def body(x_ref, w1_ref, w2_ref, out_ref, h_ref, g_ref, w2b_ref,
             sbuf_a, sbuf_b, recv_a, recv_b,
             rsa_send, rsa_recv, rsb_send, rsb_recv,
             aga_send, aga_recv, agb_send, agb_recv):
        my = lax.axis_index("i")
        left = (my + N_DEV - 1) % N_DEV
        right = (my + 1) % N_DEV

        def row_a(c, k=0):
            return ((c % N_DEV) * chunk) + k * sub

        def row_b(c, k=0):
            return ((c % N_DEV) * chunk) + half + k * sub

        def gemm1_tile(row, rows=half):
            h_ref[pl.ds(row, rows), :] = jnp.dot(
                x_ref[pl.ds(row, rows), :], w1_ref[...],
                preferred_element_type=jnp.float32,
            )

        def gemm2_piece(row):
            out_ref[pl.ds(row, sub), :] = jnp.dot(
                g_ref[pl.ds(row, sub), :], w2b_ref[...],
                preferred_element_type=jnp.float32,
            )

        def rs_send(buf, row, slot, dst_dev, sems_s, sems_r, rbuf):
            buf[slot, :, :] = h_ref[pl.ds(row, sub), :].astype(jnp.bfloat16)
            rdma = pltpu.make_async_remote_copy(
                src_ref=buf.at[slot],
                dst_ref=rbuf.at[slot],
                send_sem=sems_s.at[slot],
                recv_sem=sems_r.at[slot],
                device_id=(dst_dev,),
                device_id_type=pl.DeviceIdType.MESH,
            )
            rdma.start()
            return rdma

        def ag_send(row, slot, dst_dev, sems_s, sems_r):
            rdma = pltpu.make_async_remote_copy(
                src_ref=g_ref.at[pl.ds(row, sub)],
                dst_ref=g_ref.at[pl.ds(row, sub)],
                send_sem=sems_s.at[slot],
                recv_sem=sems_r.at[slot],
                device_id=(dst_dev,),
                device_id_type=pl.DeviceIdType.MESH,
            )
            rdma.start()
            return rdma

        barrier_sem = pltpu.get_barrier_semaphore()
        for nbr in [left, right]:
            pl.semaphore_signal(
                barrier_sem, inc=1,
                device_id=(nbr,), device_id_type=pl.DeviceIdType.MESH,
            )
        w2b_ref[...] = w2_ref[...].astype(jnp.bfloat16)
        gemm1_tile(row_a(my + 3, 0), sub)
        pl.semaphore_wait(barrier_sem, 2)

        ra = [None] * nslots
        rb = [None] * nslots
        ra[0] = rs_send(sbuf_a, row_a(my + 3, 0), 0, right,
                        rsa_send, rsa_recv, recv_a)
        gemm1_tile(row_b(my + 1, 0), sub)
        rb[0] = rs_send(sbuf_b, row_b(my + 1, 0), 0, left,
                        rsb_send, rsb_recv, recv_b)
        for k in range(1, SUB):
            gemm1_tile(row_a(my + 3, k), sub)
            ra[k] = rs_send(sbuf_a, row_a(my + 3, k), k, right,
                            rsa_send, rsa_recv, recv_a)
            gemm1_tile(row_b(my + 1, k), sub)
            rb[k] = rs_send(sbuf_b, row_b(my + 1, k), k, left,
                            rsb_send, rsb_recv, recv_b)
        gemm1_tile(row_a(my + 2))
        gemm1_tile(row_b(my + 2))
        gemm1_tile(row_a(my + 1))
        gemm1_tile(row_b(my + 3))
        gemm1_tile(row_a(my))
        gemm1_tile(row_b(my))

        ga = [None] * nslots
        gb = [None] * nslots
        for s in range(N_DEV - 1):
            c_ra = my + 2 * N_DEV - 2 - s
            c_rb = my + 2 + s
            for k in range(SUB):
                slot = s * SUB + k
                ra[slot].wait()
                h_ref[pl.ds(row_a(c_ra, k), sub), :] += (
                    recv_a[slot].astype(jnp.float32))
                if s < N_DEV - 2:
                    ra[slot + SUB] = rs_send(
                        sbuf_a, row_a(c_ra, k), slot + SUB, right,
                        rsa_send, rsa_recv, recv_a)
                else:
                    g_ref[pl.ds(row_a(my, k), sub), :] = (
                        h_ref[pl.ds(row_a(my, k), sub), :]
                        .astype(jnp.bfloat16))
                    ga[k] = ag_send(row_a(my, k), k, right,
                                    aga_send, aga_recv)
                rb[slot].wait()
                h_ref[pl.ds(row_b(c_rb, k), sub), :] += (
                    recv_b[slot].astype(jnp.float32))
                if s < N_DEV - 2:
                    rb[slot + SUB] = rs_send(
                        sbuf_b, row_b(c_rb, k), slot + SUB, left,
                        rsb_send, rsb_recv, recv_b)
                else:
                    g_ref[pl.ds(row_b(my, k), sub), :] = (
                        h_ref[pl.ds(row_b(my, k), sub), :]
                        .astype(jnp.bfloat16))
                    gb[k] = ag_send(row_b(my, k), k, left,
                                    agb_send, agb_recv)

        for k in range(SUB):
            gemm2_piece(row_a(my, k))
            gemm2_piece(row_b(my, k))

        for hh in range(N_DEV - 1):
            c_ga = my + 2 * N_DEV - 1 - hh
            c_gb = my + 1 + hh
            for k in range(SUB):
                slot = hh * SUB + k
                ga[slot].wait()
                if hh < N_DEV - 2:
                    ga[slot + SUB] = ag_send(row_a(c_ga, k), slot + SUB,
                                             right, aga_send, aga_recv)
                gemm2_piece(row_a(c_ga, k))
                gb[slot].wait()
                if hh < N_DEV - 2:
                    gb[slot + SUB] = ag_send(row_b(c_gb, k), slot + SUB,
                                             left, agb_send, agb_recv)
                gemm2_piece(row_b(c_gb, k))

    return pl.pallas_call(
        body,
        out_shape=jax.ShapeDtypeStruct((m, n), jnp.float32),
        in_specs=[
            pl.BlockSpec(memory_space=pltpu.VMEM),
            pl.BlockSpec(memory_space=pltpu.VMEM),
            pl.BlockSpec(memory_space=pltpu.VMEM),
        ],
        out_specs=pl.BlockSpec(memory_space=pltpu.VMEM),
        scratch_shapes=[
            pltpu.VMEM((m, d), jnp.float32),
            pltpu.VMEM((m, d), jnp.bfloat16),
            pltpu.VMEM((d, n), jnp.bfloat16),
            pltpu.VMEM((NSLOTS, half // SUB, d), jnp.bfloat16),
            pltpu.VMEM((NSLOTS, half // SUB, d), jnp.bfloat16),
            pltpu.VMEM((NSLOTS, half // SUB, d), jnp.bfloat16),
            pltpu.VMEM((NSLOTS, half // SUB, d), jnp.bfloat16),
            pltpu.SemaphoreType.DMA((NSLOTS,)),
            pltpu.SemaphoreType.DMA((NSLOTS,)),
            pltpu.SemaphoreType.DMA((NSLOTS,)),
            pltpu.SemaphoreType.DMA((NSLOTS,)),
            pltpu.SemaphoreType.DMA((NSLOTS,)),
            pltpu.SemaphoreType.DMA((NSLOTS,)),
            pltpu.SemaphoreType.DMA((NSLOTS,)),
            pltpu.SemaphoreType.DMA((NSLOTS,)),
        ],
        compiler_params=pltpu.CompilerParams(collective_id=0),
    )(x, W1, W2)
